# baseline (speedup 1.0000x reference)
import jax
import jax.numpy as jnp
from jax.experimental import pallas as pl
from jax.experimental.pallas import tpu as pltpu

_CHUNK = 256
_NBUF = 8


def _router(x_ref, w1_ref, b1_ref, w2_ref, b2_ref, alpha_ref, logits_ref,
            xbuf, sems):
    n_chunks = x_ref.shape[0] // _CHUNK

    def copy(slot, row_start):
        return pltpu.make_async_copy(
            x_ref.at[pl.ds(row_start, _CHUNK), :],
            xbuf.at[slot],
            sems.at[slot],
        )

    for s in range(_NBUF):
        copy(s, s * _CHUNK).start()

    w1 = w1_ref[...]
    b1 = b1_ref[...][None, :]
    w2 = w2_ref[...]
    b2 = b2_ref[...][None, :]

    def outer(o, _):
        base = o * _NBUF
        for s in range(_NBUF):
            c = base + s
            copy(s, c * _CHUNK).wait()

            h = jnp.dot(xbuf[s], w1, preferred_element_type=jnp.float32) + b1
            h = jnp.where(h > 0, h, jnp.exp(jnp.minimum(h, 0.0)) - 1.0)
            logits = jnp.dot(h, w2, preferred_element_type=jnp.float32) + b2

            n, e = logits.shape
            j = jax.lax.broadcasted_iota(jnp.int32, (n, e), 1)
            m1 = jnp.max(logits, axis=1, keepdims=True)
            idx1 = jnp.min(jnp.where(logits == m1, j, e), axis=1, keepdims=True)
            keep1 = j == idx1
            rest = jnp.where(keep1, jnp.float32(-jnp.inf), logits)
            m2 = jnp.max(rest, axis=1, keepdims=True)
            idx2 = jnp.min(jnp.where(rest == m2, j, e), axis=1, keepdims=True)
            keep = keep1 | (j == idx2)
            e_val = jnp.where(keep, jnp.exp(logits - m1), 0.0)
            alpha = e_val / jnp.sum(e_val, axis=1, keepdims=True)

            logits_ref[pl.ds(c * _CHUNK, _CHUNK), :] = logits
            alpha_ref[pl.ds(c * _CHUNK, _CHUNK), :] = alpha

            nxt = c + _NBUF

            @pl.when(nxt < n_chunks)
            def _():
                copy(s, nxt * _CHUNK).start()

        return _

    jax.lax.fori_loop(0, n_chunks // _NBUF, outer, None)


@jax.jit
def kernel(x, W1, b1, W2, b2):
    n_tokens = x.shape[0]
    n_exp = W2.shape[1]
    alpha, logits = pl.pallas_call(
        _router,
        in_specs=[
            pl.BlockSpec(memory_space=pltpu.HBM),
            pl.BlockSpec(memory_space=pltpu.VMEM),
            pl.BlockSpec(memory_space=pltpu.VMEM),
            pl.BlockSpec(memory_space=pltpu.VMEM),
            pl.BlockSpec(memory_space=pltpu.VMEM),
        ],
        out_specs=[
            pl.BlockSpec(memory_space=pltpu.VMEM),
            pl.BlockSpec(memory_space=pltpu.VMEM),
        ],
        out_shape=[
            jax.ShapeDtypeStruct((n_tokens, n_exp), jnp.float32),
            jax.ShapeDtypeStruct((n_tokens, n_exp), jnp.float32),
        ],
        scratch_shapes=[
            pltpu.VMEM((_NBUF, _CHUNK, x.shape[1]), jnp.float32),
            pltpu.SemaphoreType.DMA((_NBUF,)),
        ],
    )(x, W1, b1, W2, b2)
    return alpha, logits


# speedup vs baseline: 1.4138x; 1.4138x over previous
import jax
import jax.numpy as jnp
from jax.experimental import pallas as pl
from jax.experimental.pallas import tpu as pltpu

_CHUNK = 512
_NBUF = 8
_GSLOTS = 2


def _router(x_ref, w1_ref, b1_ref, w2_ref, b2_ref, alpha_ref, logits_ref,
            xbuf, sems):
    in_dim = x_ref.shape[1]
    n_chunks = x_ref.shape[0] // _CHUNK
    rows_g = _CHUNK * _GSLOTS
    groups_per_iter = _NBUF // _GSLOTS
    n_outer = n_chunks // _NBUF

    def copy(slot, row_start):
        return pltpu.make_async_copy(
            x_ref.at[pl.ds(row_start, _CHUNK), :],
            xbuf.at[slot],
            sems.at[slot],
        )

    for s in range(_NBUF):
        copy(s, s * _CHUNK).start()

    w1 = w1_ref[...]
    b1 = b1_ref[...][None, :]
    w2 = w2_ref[...]
    b2 = b2_ref[...][None, :]

    def outer(o, _):
        for gi in range(groups_per_iter):
            s0 = gi * _GSLOTS
            c0 = o * _NBUF + s0
            for k in range(_GSLOTS):
                copy(s0 + k, (c0 + k) * _CHUNK).wait()

            xg = xbuf[s0:s0 + _GSLOTS].reshape(rows_g, in_dim)
            h = jnp.dot(xg, w1, preferred_element_type=jnp.float32) + b1
            h = jnp.where(h > 0, h, jnp.exp(jnp.minimum(h, 0.0)) - 1.0)
            logits = jnp.dot(h, w2, preferred_element_type=jnp.float32) + b2

            n, e = logits.shape
            j = jax.lax.broadcasted_iota(jnp.int32, (n, e), 1)
            m1 = jnp.max(logits, axis=1, keepdims=True)
            idx1 = jnp.min(jnp.where(logits == m1, j, e), axis=1, keepdims=True)
            keep1 = j == idx1
            rest = jnp.where(keep1, jnp.float32(-jnp.inf), logits)
            m2 = jnp.max(rest, axis=1, keepdims=True)
            idx2 = jnp.min(jnp.where(rest == m2, j, e), axis=1, keepdims=True)
            keep = keep1 | (j == idx2)
            e_val = jnp.where(keep, jnp.exp(logits - m1), 0.0)
            alpha = e_val / jnp.sum(e_val, axis=1, keepdims=True)

            row0 = c0 * _CHUNK
            logits_ref[pl.ds(row0, rows_g), :] = logits
            alpha_ref[pl.ds(row0, rows_g), :] = alpha

            for k in range(_GSLOTS):
                nxt = c0 + k + _NBUF

                @pl.when(nxt < n_chunks)
                def _():
                    copy(s0 + k, nxt * _CHUNK).start()

        return _

    jax.lax.fori_loop(0, n_outer, outer, None)


@jax.jit
def kernel(x, W1, b1, W2, b2):
    n_tokens = x.shape[0]
    n_exp = W2.shape[1]
    alpha, logits = pl.pallas_call(
        _router,
        in_specs=[
            pl.BlockSpec(memory_space=pltpu.HBM),
            pl.BlockSpec(memory_space=pltpu.VMEM),
            pl.BlockSpec(memory_space=pltpu.VMEM),
            pl.BlockSpec(memory_space=pltpu.VMEM),
            pl.BlockSpec(memory_space=pltpu.VMEM),
        ],
        out_specs=[
            pl.BlockSpec(memory_space=pltpu.VMEM),
            pl.BlockSpec(memory_space=pltpu.VMEM),
        ],
        out_shape=[
            jax.ShapeDtypeStruct((n_tokens, n_exp), jnp.float32),
            jax.ShapeDtypeStruct((n_tokens, n_exp), jnp.float32),
        ],
        scratch_shapes=[
            pltpu.VMEM((_NBUF, _CHUNK, x.shape[1]), jnp.float32),
            pltpu.SemaphoreType.DMA((_NBUF,)),
        ],
    )(x, W1, b1, W2, b2)
    return alpha, logits


# speedup vs baseline: 1.5929x; 1.1267x over previous
import jax
import jax.numpy as jnp
from jax.experimental import pallas as pl
from jax.experimental.pallas import tpu as pltpu

_BLOCK = 2048
_RSPLIT = 2
_RCHUNK = _BLOCK // _RSPLIT


def _router_block(*refs):
    x_refs = refs[:_RSPLIT]
    w1_ref, b1_ref, w2_ref, b2_ref, alpha_ref, logits_ref = refs[_RSPLIT:]

    w1 = w1_ref[...]
    b1 = b1_ref[...][None, :]
    w2 = w2_ref[...]
    b2 = b2_ref[...][None, :]

    for r in range(_RSPLIT):
        h = jnp.dot(x_refs[r][...], w1, preferred_element_type=jnp.float32) + b1
        h = jnp.where(h > 0, h, jnp.exp(jnp.minimum(h, 0.0)) - 1.0)
        logits = jnp.dot(h, w2, preferred_element_type=jnp.float32) + b2

        n, e = logits.shape
        j = jax.lax.broadcasted_iota(jnp.int32, (n, e), 1)
        m1 = jnp.max(logits, axis=1, keepdims=True)
        idx1 = jnp.min(jnp.where(logits == m1, j, e), axis=1, keepdims=True)
        keep1 = j == idx1
        rest = jnp.where(keep1, jnp.float32(-jnp.inf), logits)
        m2 = jnp.max(rest, axis=1, keepdims=True)
        idx2 = jnp.min(jnp.where(rest == m2, j, e), axis=1, keepdims=True)
        keep = keep1 | (j == idx2)
        e_val = jnp.where(keep, jnp.exp(logits - m1), 0.0)
        alpha = e_val / jnp.sum(e_val, axis=1, keepdims=True)

        logits_ref[pl.ds(r * _RCHUNK, _RCHUNK), :] = logits
        alpha_ref[pl.ds(r * _RCHUNK, _RCHUNK), :] = alpha


def _x_spec(r):
    return pl.BlockSpec((_RCHUNK, 2048), lambda i, r=r: (i * _RSPLIT + r, 0))


@jax.jit
def kernel(x, W1, b1, W2, b2):
    n_tokens, in_dim = x.shape
    hidden = W1.shape[1]
    n_exp = W2.shape[1]
    grid = (n_tokens // _BLOCK,)
    alpha, logits = pl.pallas_call(
        _router_block,
        grid=grid,
        in_specs=[_x_spec(r) for r in range(_RSPLIT)]
        + [
            pl.BlockSpec((in_dim, hidden), lambda i: (0, 0)),
            pl.BlockSpec((hidden,), lambda i: (0,)),
            pl.BlockSpec((hidden, n_exp), lambda i: (0, 0)),
            pl.BlockSpec((n_exp,), lambda i: (0,)),
        ],
        out_specs=[
            pl.BlockSpec((_BLOCK, n_exp), lambda i: (i, 0)),
            pl.BlockSpec((_BLOCK, n_exp), lambda i: (i, 0)),
        ],
        out_shape=[
            jax.ShapeDtypeStruct((n_tokens, n_exp), jnp.float32),
            jax.ShapeDtypeStruct((n_tokens, n_exp), jnp.float32),
        ],
        compiler_params=pltpu.CompilerParams(
            dimension_semantics=(pltpu.PARALLEL,),
        ),
    )(*([x] * _RSPLIT), W1, b1, W2, b2)
    return alpha, logits


# speedup vs baseline: 1.6811x; 1.0554x over previous
import jax
import jax.numpy as jnp
from jax.experimental import pallas as pl
from jax.experimental.pallas import tpu as pltpu

_BLOCK = 2048


def _router_block(x_ref, w1_ref, b1_ref, w2_ref, b2_ref, alpha_ref, logits_ref):
    h = jnp.dot(x_ref[...], w1_ref[...], preferred_element_type=jnp.float32)
    h = h + b1_ref[...]
    h = jnp.where(h > 0, h, jnp.exp(jnp.minimum(h, 0.0)) - 1.0)
    logits = jnp.dot(h, w2_ref[...], preferred_element_type=jnp.float32)
    logits = logits + b2_ref[...]

    n, e = logits.shape
    j = jax.lax.broadcasted_iota(jnp.int32, (n, e), 1)
    neg_inf = jnp.float32(-jnp.inf)

    m1 = jnp.max(logits, axis=1, keepdims=True)
    idx1 = jnp.min(jnp.where(logits == m1, j, e), axis=1, keepdims=True)
    keep1 = j == idx1

    rest = jnp.where(keep1, neg_inf, logits)
    m2 = jnp.max(rest, axis=1, keepdims=True)
    idx2 = jnp.min(jnp.where(rest == m2, j, e), axis=1, keepdims=True)
    keep = keep1 | (j == idx2)

    e_val = jnp.where(keep, jnp.exp(logits - m1), 0.0)
    alpha = e_val / jnp.sum(e_val, axis=1, keepdims=True)

    logits_ref[...] = logits
    alpha_ref[...] = alpha


@jax.jit
def kernel(x, W1, b1, W2, b2):
    n_tokens, in_dim = x.shape
    hidden = W1.shape[1]
    n_exp = W2.shape[1]
    grid = (n_tokens // _BLOCK,)
    alpha, logits = pl.pallas_call(
        _router_block,
        grid=grid,
        in_specs=[
            pl.BlockSpec((_BLOCK, in_dim), lambda i: (i, 0)),
            pl.BlockSpec((in_dim, hidden), lambda i: (0, 0)),
            pl.BlockSpec((hidden,), lambda i: (0,)),
            pl.BlockSpec((hidden, n_exp), lambda i: (0, 0)),
            pl.BlockSpec((n_exp,), lambda i: (0,)),
        ],
        out_specs=[
            pl.BlockSpec((_BLOCK, n_exp), lambda i: (i, 0)),
            pl.BlockSpec((_BLOCK, n_exp), lambda i: (i, 0)),
        ],
        out_shape=[
            jax.ShapeDtypeStruct((n_tokens, n_exp), jnp.float32),
            jax.ShapeDtypeStruct((n_tokens, n_exp), jnp.float32),
        ],
        compiler_params=pltpu.CompilerParams(
            dimension_semantics=(pltpu.PARALLEL,),
        ),
    )(x, W1, b1, W2, b2)
    return alpha, logits


# speedup vs baseline: 1.6812x; 1.0000x over previous
import jax
import jax.numpy as jnp
from jax.experimental import pallas as pl
from jax.experimental.pallas import tpu as pltpu

_BLOCK = 2048


def _router_block(xa_ref, xb_ref, w1_ref, b1_ref, w2_ref, b2_ref,
                  alpha_ref, logits_ref):
    kc = w1_ref.shape[0] // 2
    h = jnp.dot(xa_ref[...], w1_ref[:kc, :], preferred_element_type=jnp.float32)
    h = h + jnp.dot(xb_ref[...], w1_ref[kc:, :], preferred_element_type=jnp.float32)
    h = h + b1_ref[...]
    h = jnp.where(h > 0, h, jnp.exp(jnp.minimum(h, 0.0)) - 1.0)
    logits = jnp.dot(h, w2_ref[...], preferred_element_type=jnp.float32)
    logits = logits + b2_ref[...]

    n, e = logits.shape
    j = jax.lax.broadcasted_iota(jnp.int32, (n, e), 1)
    neg_inf = jnp.float32(-jnp.inf)

    m1 = jnp.max(logits, axis=1, keepdims=True)
    idx1 = jnp.min(jnp.where(logits == m1, j, e), axis=1, keepdims=True)
    keep1 = j == idx1

    rest = jnp.where(keep1, neg_inf, logits)
    m2 = jnp.max(rest, axis=1, keepdims=True)
    idx2 = jnp.min(jnp.where(rest == m2, j, e), axis=1, keepdims=True)
    keep = keep1 | (j == idx2)

    e_val = jnp.where(keep, jnp.exp(logits - m1), 0.0)
    alpha = e_val / jnp.sum(e_val, axis=1, keepdims=True)

    logits_ref[...] = logits
    alpha_ref[...] = alpha


@jax.jit
def kernel(x, W1, b1, W2, b2):
    n_tokens, in_dim = x.shape
    hidden = W1.shape[1]
    n_exp = W2.shape[1]
    grid = (n_tokens // _BLOCK,)
    alpha, logits = pl.pallas_call(
        _router_block,
        grid=grid,
        in_specs=[
            pl.BlockSpec((_BLOCK, in_dim // 2), lambda i: (i, 0)),
            pl.BlockSpec((_BLOCK, in_dim // 2), lambda i: (i, 1)),
            pl.BlockSpec((in_dim, hidden), lambda i: (0, 0)),
            pl.BlockSpec((hidden,), lambda i: (0,)),
            pl.BlockSpec((hidden, n_exp), lambda i: (0, 0)),
            pl.BlockSpec((n_exp,), lambda i: (0,)),
        ],
        out_specs=[
            pl.BlockSpec((_BLOCK, n_exp), lambda i: (i, 0)),
            pl.BlockSpec((_BLOCK, n_exp), lambda i: (i, 0)),
        ],
        out_shape=[
            jax.ShapeDtypeStruct((n_tokens, n_exp), jnp.float32),
            jax.ShapeDtypeStruct((n_tokens, n_exp), jnp.float32),
        ],
        compiler_params=pltpu.CompilerParams(
            dimension_semantics=(pltpu.PARALLEL,),
        ),
    )(x, x, W1, b1, W2, b2)
    return alpha, logits


# speedup vs baseline: 1.6898x; 1.0051x over previous
import jax
import jax.numpy as jnp
from jax.experimental import pallas as pl
from jax.experimental.pallas import tpu as pltpu

_BLOCK = 2048


def _router_block(x_ref, w1_ref, b1_ref, w2_ref, b2_ref, alpha_ref, logits_ref):
    h = jnp.dot(x_ref[...], w1_ref[...], preferred_element_type=jnp.float32)
    h = h + b1_ref[...]
    h = jnp.where(h > 0, h, jnp.exp(jnp.minimum(h, 0.0)) - 1.0)
    logits = jnp.dot(h, w2_ref[...], preferred_element_type=jnp.float32)
    logits = logits + b2_ref[...]

    n, e = logits.shape
    j = jax.lax.broadcasted_iota(jnp.int32, (n, e), 1)
    neg_inf = jnp.float32(-jnp.inf)

    m1 = jnp.max(logits, axis=1, keepdims=True)
    idx1 = jnp.min(jnp.where(logits == m1, j, e), axis=1, keepdims=True)
    keep1 = j == idx1

    rest = jnp.where(keep1, neg_inf, logits)
    m2 = jnp.max(rest, axis=1, keepdims=True)
    idx2 = jnp.min(jnp.where(rest == m2, j, e), axis=1, keepdims=True)
    keep = keep1 | (j == idx2)

    e_val = jnp.where(keep, jnp.exp(logits - m1), 0.0)
    alpha = e_val / jnp.sum(e_val, axis=1, keepdims=True)

    logits_ref[...] = logits
    alpha_ref[...] = alpha


@jax.jit
def kernel(x, W1, b1, W2, b2):
    n_tokens, in_dim = x.shape
    hidden = W1.shape[1]
    n_exp = W2.shape[1]
    grid = (n_tokens // _BLOCK,)
    alpha, logits = pl.pallas_call(
        _router_block,
        grid=grid,
        in_specs=[
            pl.BlockSpec((_BLOCK, in_dim), lambda i: (i, 0)),
            pl.BlockSpec((in_dim, hidden), lambda i: (0, 0)),
            pl.BlockSpec((hidden,), lambda i: (0,)),
            pl.BlockSpec((hidden, n_exp), lambda i: (0, 0)),
            pl.BlockSpec((n_exp,), lambda i: (0,)),
        ],
        out_specs=[
            pl.BlockSpec((_BLOCK, n_exp), lambda i: (i, 0)),
            pl.BlockSpec((_BLOCK, n_exp), lambda i: (i, 0)),
        ],
        out_shape=[
            jax.ShapeDtypeStruct((n_tokens, n_exp), jnp.float32),
            jax.ShapeDtypeStruct((n_tokens, n_exp), jnp.float32),
        ],
        compiler_params=pltpu.CompilerParams(
            dimension_semantics=(pltpu.PARALLEL,),
        ),
    )(x, W1, b1, W2, b2)
    return alpha, logits
